# Initial kernel scaffold; baseline (speedup 1.0000x reference)
import jax
import jax.numpy as jnp
from jax.experimental import pallas as pl


def kernel(data):
    raise NotImplementedError("write your pallas kernel here")


# speedup vs baseline: 1.1445x; 1.1445x over previous
import functools

import jax
import jax.numpy as jnp
from jax import lax
from jax.experimental import pallas as pl
from jax.experimental.pallas import tpu as pltpu
from jax.experimental.pallas import tpu_sc as plsc

NC, NS, L = 2, 16, 16
NW = NC * NS
B, C, N = 128, 512, 128
HALF = N // 2
ROWS = B * C
RPW = ROWS // NW
CHUNK = 256
NCHUNK = RPW // CHUNK

_mesh = plsc.VectorSubcoreMesh(core_axis_name="c", subcore_axis_name="s")


@functools.partial(
    pl.kernel,
    mesh=_mesh,
    out_type=jax.ShapeDtypeStruct((ROWS * HALF,), jnp.float32),
    scratch_types=[
        pltpu.VMEM((CHUNK * N,), jnp.float32),
        pltpu.VMEM((CHUNK * HALF,), jnp.float32),
    ],
)
def _sc_maxpool(x_hbm, o_hbm, xv, ov):
    wid = lax.axis_index("s") * NC + lax.axis_index("c")
    base_row = wid * RPW

    def chunk_body(ci, carry):
        row0 = base_row + ci * CHUNK
        pltpu.sync_copy(x_hbm.at[pl.ds(row0 * N, CHUNK * N)], xv)

        def row_body(r, c2):
            for kk in range(N // (2 * L)):
                a = xv[pl.ds(r * N + kk * L, L)]
                b = xv[pl.ds(r * N + HALF + kk * L, L)]
                ov[pl.ds(r * HALF + kk * L, L)] = jnp.maximum(a, b)
            return c2

        lax.fori_loop(0, CHUNK, row_body, 0)
        pltpu.sync_copy(ov, o_hbm.at[pl.ds(row0 * HALF, CHUNK * HALF)])
        return carry

    lax.fori_loop(0, NCHUNK, chunk_body, 0)


def kernel(data):
    x = data.reshape(ROWS * N)
    out = _sc_maxpool(x)
    return out.reshape(B, C, HALF)


# speedup vs baseline: 1.6540x; 1.4452x over previous
import functools

import jax
import jax.numpy as jnp
from jax import lax
from jax.experimental import pallas as pl
from jax.experimental.pallas import tpu as pltpu
from jax.experimental.pallas import tpu_sc as plsc

NC, NS, L = 2, 16, 16
NW = NC * NS
B, C, N = 128, 512, 128
HALF = N // 2
ROWS = B * C
RPW = ROWS // NW
CHUNK = 256
NCHUNK = RPW // CHUNK

_mesh = plsc.VectorSubcoreMesh(core_axis_name="c", subcore_axis_name="s")


@functools.partial(
    pl.kernel,
    mesh=_mesh,
    out_type=jax.ShapeDtypeStruct((ROWS * HALF,), jnp.float32),
    scratch_types=[
        pltpu.VMEM((CHUNK * N,), jnp.float32),
        pltpu.VMEM((CHUNK * N,), jnp.float32),
        pltpu.VMEM((CHUNK * HALF,), jnp.float32),
        pltpu.VMEM((CHUNK * HALF,), jnp.float32),
        pltpu.SemaphoreType.DMA,
        pltpu.SemaphoreType.DMA,
        pltpu.SemaphoreType.DMA,
        pltpu.SemaphoreType.DMA,
    ],
)
def _sc_maxpool(x_hbm, o_hbm, xv0, xv1, ov0, ov1, is0, is1, os0, os1):
    wid = lax.axis_index("s") * NC + lax.axis_index("c")
    base_row = wid * RPW
    xb, ob, isem, osem = (xv0, xv1), (ov0, ov1), (is0, is1), (os0, os1)

    def start_in(ci):
        b = ci % 2
        return pltpu.async_copy(
            x_hbm.at[pl.ds((base_row + ci * CHUNK) * N, CHUNK * N)],
            xb[b], isem[b])

    def start_out(ci):
        b = ci % 2
        return pltpu.async_copy(
            ob[b],
            o_hbm.at[pl.ds((base_row + ci * CHUNK) * HALF, CHUNK * HALF)],
            osem[b])

    def compute(ci):
        xv, ov = xb[ci % 2], ob[ci % 2]

        @plsc.parallel_loop(0, CHUNK, 1, unroll=4)
        def row_body(r):
            for kk in range(HALF // L):
                a = xv[pl.ds(r * N + kk * L, L)]
                b2 = xv[pl.ds(r * N + HALF + kk * L, L)]
                ov[pl.ds(r * HALF + kk * L, L)] = jnp.maximum(a, b2)

    in_d = {0: start_in(0), 1: start_in(1)}
    out_d = {}
    for ci in range(NCHUNK):
        in_d[ci].wait()
        if ci >= 2:
            out_d[ci - 2].wait()
        compute(ci)
        out_d[ci] = start_out(ci)
        if ci + 2 < NCHUNK:
            in_d[ci + 2] = start_in(ci + 2)
    out_d[NCHUNK - 2].wait()
    out_d[NCHUNK - 1].wait()


def kernel(data):
    x = data.reshape(ROWS * N)
    out = _sc_maxpool(x)
    return out.reshape(B, C, HALF)
